# baseline (speedup 1.0000x reference)
import functools

import jax
import jax.numpy as jnp
from jax import lax
from jax.experimental import pallas as pl
from jax.experimental.pallas import tpu as pltpu
from jax.experimental.pallas import tpu_sc as plsc

EMBED_DIM = 32
HIST = 200
BATCH = 16384
_NC, _NS = 2, 16
_NW = _NC * _NS
_BB = 512
_BLK = BATCH // _BB
_NITEM = HIST * _BLK
_PER_W = _NITEM // _NW
_NPAIR = _PER_W // 2
_TW = _BB * EMBED_DIM // 4
_OW = BATCH * 8

_mesh = plsc.VectorSubcoreMesh(core_axis_name="c", subcore_axis_name="s")


@functools.partial(
    pl.kernel,
    mesh=_mesh,
    out_type=jax.ShapeDtypeStruct((HIST, 4, _OW), jnp.float32),
    scratch_types=[
        pltpu.VMEM((_BB,), jnp.int32),
        pltpu.VMEM((_BB,), jnp.int32),
        pltpu.VMEM((_BB, EMBED_DIM), jnp.float32),
        pltpu.VMEM((_BB, EMBED_DIM), jnp.float32),
        pltpu.VMEM((4 * _TW,), jnp.float32),
        pltpu.VMEM((4 * _TW,), jnp.float32),
        pltpu.SemaphoreType.DMA,
        pltpu.SemaphoreType.DMA,
        pltpu.SemaphoreType.DMA,
        pltpu.SemaphoreType.DMA,
        pltpu.SemaphoreType.DMA,
        pltpu.SemaphoreType.DMA,
    ],
    compiler_params=pltpu.CompilerParams(
        use_tc_tiling_on_sc=False, needs_layout_passes=False),
)
def _gather_kernel(ids_hbm, table_hbm, out_hbm,
                   idx0, idx1, g0, g1, t0, t1,
                   si0, si1, sg0, sg1, so0, so1):
    wid = lax.axis_index("s") * _NC + lax.axis_index("c")
    wbase = wid * _PER_W
    iota = lax.iota(jnp.int32, 16)
    pat = (iota >> 3) * _TW + (iota & 7) * 128

    def idx_load(n, idx_v, sem):
        pltpu.async_copy(ids_hbm.at[pl.ds((wbase + n) * _BB, _BB)], idx_v, sem)

    def idx_wait(n, idx_v, sem):
        pltpu.make_async_copy(
            ids_hbm.at[pl.ds((wbase + n) * _BB, _BB)], idx_v, sem).wait()

    def gather(idx_v, g_v, sem):
        pltpu.async_copy(table_hbm.at[idx_v], g_v, sem)

    def gather_wait(idx_v, g_v, sem):
        pltpu.make_async_copy(table_hbm.at[idx_v], g_v, sem).wait()

    def out_store(n, t_v, sem):
        item = wbase + n
        h = item >> 5
        blk = item & 31
        for d8 in range(4):
            pltpu.async_copy(t_v.at[pl.ds(d8 * _TW, _TW)],
                             out_hbm.at[h, d8, pl.ds(blk * _TW, _TW)], sem)

    def out_wait(n, t_v, sem):
        item = wbase + n
        h = item >> 5
        blk = item & 31
        for d8 in range(4):
            pltpu.make_async_copy(
                t_v.at[pl.ds(d8 * _TW, _TW)],
                out_hbm.at[h, d8, pl.ds(blk * _TW, _TW)], sem).wait()

    def transpose(g_v, t_v):
        @plsc.parallel_loop(0, 128, 1, unroll=4)
        def tbody(u):
            bl = (u >> 3) & 3
            col = jnp.zeros((16,), jnp.int32) + (((u >> 5) << 3) | (u & 7))
            tb = u << 7
            rb = bl * 128
            for b16 in range(8):
                rows = iota + (rb + b16 * 16)
                v = plsc.load_gather(g_v, [rows, col])
                t_v[pl.ds(tb + b16 * 16, 16)] = v

    idx_load(0, idx0, si0)
    idx_wait(0, idx0, si0)
    gather(idx0, g0, sg0)
    idx_load(1, idx1, si1)

    def pair(j, carry):
        a = 2 * j
        b = a + 1
        gather_wait(idx0, g0, sg0)

        @pl.when(j < _NPAIR - 1)
        def _():
            idx_load(a + 2, idx0, si0)

        idx_wait(b, idx1, si1)
        gather(idx1, g1, sg1)

        @pl.when(j > 0)
        def _():
            out_wait(a - 2, t0, so0)
        transpose(g0, t0)
        out_store(a, t0, so0)

        gather_wait(idx1, g1, sg1)

        @pl.when(j < _NPAIR - 1)
        def _():
            idx_load(b + 2, idx1, si1)
            idx_wait(a + 2, idx0, si0)
            gather(idx0, g0, sg0)

        @pl.when(j > 0)
        def _():
            out_wait(b - 2, t1, so1)
        transpose(g1, t1)
        out_store(b, t1, so1)
        return carry

    lax.fori_loop(0, _NPAIR, pair, 0)

    out_wait(_PER_W - 2, t0, so0)
    out_wait(_PER_W - 1, t1, so1)


def kernel(ids, embs, pad):
    del pad
    flat_t = ids.T.reshape(-1).astype(jnp.int32)
    out3 = _gather_kernel(flat_t, embs)
    out5 = out3.reshape(HIST, 4, 128, 8, 128)
    return out5.transpose(2, 4, 0, 1, 3).reshape(BATCH, HIST, EMBED_DIM)


# speedup vs baseline: 1.0285x; 1.0285x over previous
import functools

import jax
import jax.numpy as jnp
from jax import lax
from jax.experimental import pallas as pl
from jax.experimental.pallas import tpu as pltpu
from jax.experimental.pallas import tpu_sc as plsc

EMBED_DIM = 32
HIST = 200
BATCH = 16384
_NC, _NS = 2, 16
_NW = _NC * _NS
_BB = 512
_BLK = BATCH // _BB
_NITEM = HIST * _BLK
_PER_W = _NITEM // _NW
_NPAIR = _PER_W // 2
_TW = _BB * EMBED_DIM // 4
_OW = BATCH * 8

_mesh = plsc.VectorSubcoreMesh(core_axis_name="c", subcore_axis_name="s")


@functools.partial(
    pl.kernel,
    mesh=_mesh,
    out_type=jax.ShapeDtypeStruct((HIST, 4, _OW), jnp.float32),
    scratch_types=[
        pltpu.VMEM((_BB,), jnp.int32),
        pltpu.VMEM((_BB,), jnp.int32),
        pltpu.VMEM((_BB, EMBED_DIM), jnp.float32),
        pltpu.VMEM((_BB, EMBED_DIM), jnp.float32),
        pltpu.VMEM((4 * _TW,), jnp.float32),
        pltpu.VMEM((4 * _TW,), jnp.float32),
        pltpu.SemaphoreType.DMA,
        pltpu.SemaphoreType.DMA,
        pltpu.SemaphoreType.DMA,
        pltpu.SemaphoreType.DMA,
        pltpu.SemaphoreType.DMA,
        pltpu.SemaphoreType.DMA,
    ],
    compiler_params=pltpu.CompilerParams(
        use_tc_tiling_on_sc=False, needs_layout_passes=False),
)
def _gather_kernel(ids_hbm, table_hbm, out_hbm,
                   idx0, idx1, g0, g1, t0, t1,
                   si0, si1, sg0, sg1, so0, so1):
    wid = lax.axis_index("s") * _NC + lax.axis_index("c")
    wbase = wid * _PER_W
    iota = lax.iota(jnp.int32, 16)
    pat = (iota >> 3) * _TW + (iota & 7) * 128

    def idx_load(n, idx_v, sem):
        pltpu.async_copy(ids_hbm.at[pl.ds((wbase + n) * _BB, _BB)], idx_v, sem)

    def idx_wait(n, idx_v, sem):
        pltpu.make_async_copy(
            ids_hbm.at[pl.ds((wbase + n) * _BB, _BB)], idx_v, sem).wait()

    def gather(idx_v, g_v, sem):
        pltpu.async_copy(table_hbm.at[idx_v], g_v, sem)

    def gather_wait(idx_v, g_v, sem):
        pltpu.make_async_copy(table_hbm.at[idx_v], g_v, sem).wait()

    def out_store(n, t_v, sem):
        item = wbase + n
        h = item >> 5
        blk = item & 31
        for d8 in range(4):
            pltpu.async_copy(t_v.at[pl.ds(d8 * _TW, _TW)],
                             out_hbm.at[h, d8, pl.ds(blk * _TW, _TW)], sem)

    def out_wait(n, t_v, sem):
        item = wbase + n
        h = item >> 5
        blk = item & 31
        for d8 in range(4):
            pltpu.make_async_copy(
                t_v.at[pl.ds(d8 * _TW, _TW)],
                out_hbm.at[h, d8, pl.ds(blk * _TW, _TW)], sem).wait()

    def transpose(g_v, t_v):
        @plsc.parallel_loop(0, 128, 1, unroll=8)
        def tbody(u):
            bl = (u >> 3) & 3
            col = jnp.zeros((16,), jnp.int32) + (((u >> 5) << 3) | (u & 7))
            tb = u << 7
            rb = bl * 128
            for b16 in range(8):
                rows = iota + (rb + b16 * 16)
                v = plsc.load_gather(g_v, [rows, col])
                t_v[pl.ds(tb + b16 * 16, 16)] = v

    idx_load(0, idx0, si0)
    idx_wait(0, idx0, si0)
    gather(idx0, g0, sg0)
    idx_load(1, idx1, si1)

    def pair(j, carry):
        a = 2 * j
        b = a + 1
        gather_wait(idx0, g0, sg0)

        @pl.when(j < _NPAIR - 1)
        def _():
            idx_load(a + 2, idx0, si0)

        idx_wait(b, idx1, si1)
        gather(idx1, g1, sg1)

        @pl.when(j > 0)
        def _():
            out_wait(a - 2, t0, so0)
        transpose(g0, t0)
        out_store(a, t0, so0)

        gather_wait(idx1, g1, sg1)

        @pl.when(j < _NPAIR - 1)
        def _():
            idx_load(b + 2, idx1, si1)
            idx_wait(a + 2, idx0, si0)
            gather(idx0, g0, sg0)

        @pl.when(j > 0)
        def _():
            out_wait(b - 2, t1, so1)
        transpose(g1, t1)
        out_store(b, t1, so1)
        return carry

    lax.fori_loop(0, _NPAIR, pair, 0)

    out_wait(_PER_W - 2, t0, so0)
    out_wait(_PER_W - 1, t1, so1)


def kernel(ids, embs, pad):
    del pad
    flat_t = ids.T.reshape(-1).astype(jnp.int32)
    out3 = _gather_kernel(flat_t, embs)
    out5 = out3.reshape(HIST, 4, 128, 8, 128)
    return out5.transpose(2, 4, 0, 1, 3).reshape(BATCH, HIST, EMBED_DIM)


# speedup vs baseline: 2.1923x; 2.1316x over previous
import functools

import jax
import jax.numpy as jnp
from jax import lax
from jax.experimental import pallas as pl
from jax.experimental.pallas import tpu as pltpu
from jax.experimental.pallas import tpu_sc as plsc

EMBED_DIM = 32
HIST = 200
BATCH = 16384
_NC, _NS = 2, 16
_NW = _NC * _NS
_BB = 512
_BLK = BATCH // _BB
_NITEM = HIST * _BLK
_PER_W = _NITEM // _NW
_NPAIR = _PER_W // 2
_TW = _BB * EMBED_DIM // 4
_OW = BATCH * 8

_mesh = plsc.VectorSubcoreMesh(core_axis_name="c", subcore_axis_name="s")


@functools.partial(
    pl.kernel,
    mesh=_mesh,
    out_type=jax.ShapeDtypeStruct((HIST, 4, _OW), jnp.float32),
    scratch_types=[
        pltpu.VMEM((_BB,), jnp.int32),
        pltpu.VMEM((_BB,), jnp.int32),
        pltpu.VMEM((_BB, EMBED_DIM), jnp.float32),
        pltpu.VMEM((_BB, EMBED_DIM), jnp.float32),
        pltpu.VMEM((4 * _TW,), jnp.float32),
        pltpu.VMEM((4 * _TW,), jnp.float32),
        pltpu.VMEM((16384,), jnp.float32),
        pltpu.SemaphoreType.DMA,
        pltpu.SemaphoreType.DMA,
        pltpu.SemaphoreType.DMA,
        pltpu.SemaphoreType.DMA,
        pltpu.SemaphoreType.DMA,
        pltpu.SemaphoreType.DMA,
    ],
    compiler_params=pltpu.CompilerParams(
        use_tc_tiling_on_sc=False, needs_layout_passes=False),
)
def _gather_kernel(ids_hbm, table_hbm, out_hbm,
                   idx0, idx1, g0, g1, t0, t1, tmp,
                   si0, si1, sg0, sg1, so0, so1):
    wid = lax.axis_index("s") * _NC + lax.axis_index("c")
    wbase = wid * _PER_W
    iota = lax.iota(jnp.int32, 16)
    pat = (iota >> 3) * _TW + (iota & 7) * 128

    def idx_load(n, idx_v, sem):
        pltpu.async_copy(ids_hbm.at[pl.ds((wbase + n) * _BB, _BB)], idx_v, sem)

    def idx_wait(n, idx_v, sem):
        pltpu.make_async_copy(
            ids_hbm.at[pl.ds((wbase + n) * _BB, _BB)], idx_v, sem).wait()

    def gather(idx_v, g_v, sem):
        pltpu.async_copy(table_hbm.at[idx_v], g_v, sem)

    def gather_wait(idx_v, g_v, sem):
        pltpu.make_async_copy(table_hbm.at[idx_v], g_v, sem).wait()

    def out_store(n, t_v, sem):
        item = wbase + n
        h = item >> 5
        blk = item & 31
        for d8 in range(4):
            pltpu.async_copy(t_v.at[pl.ds(d8 * _TW, _TW)],
                             out_hbm.at[h, d8, pl.ds(blk * _TW, _TW)], sem)

    def out_wait(n, t_v, sem):
        item = wbase + n
        h = item >> 5
        blk = item & 31
        for d8 in range(4):
            pltpu.make_async_copy(
                t_v.at[pl.ds(d8 * _TW, _TW)],
                out_hbm.at[h, d8, pl.ds(blk * _TW, _TW)], sem).wait()

    rot = [(iota + k) & 15 for k in range(16)]
    rot2 = [iota * 16 + ((iota + c) & 15) for c in range(16)]

    def transpose(g_v, t_v, tmp_v):
        @plsc.parallel_loop(0, 64, 1, unroll=4)
        def tbody(w):
            bl = w >> 4
            b16 = (w >> 1) & 7
            hh = w & 1
            b0 = bl * 128 + b16 * 16
            base_u = w << 8
            for k in range(16):
                v = g_v[b0 + k, pl.ds(hh * 16, 16)]
                plsc.store_scatter(tmp_v, [rot[k] + (base_u + k * 16)], v)
            tb = hh * 8192 + bl * 1024 + b16 * 16
            for c in range(16):
                v = plsc.load_gather(tmp_v, [rot2[c] + base_u])
                t_v[pl.ds(tb + ((c >> 3) * 4096 + (c & 7) * 128), 16)] = v

    idx_load(0, idx0, si0)
    idx_wait(0, idx0, si0)
    gather(idx0, g0, sg0)
    idx_load(1, idx1, si1)

    def pair(j, carry):
        a = 2 * j
        b = a + 1
        gather_wait(idx0, g0, sg0)

        @pl.when(j < _NPAIR - 1)
        def _():
            idx_load(a + 2, idx0, si0)

        idx_wait(b, idx1, si1)
        gather(idx1, g1, sg1)

        @pl.when(j > 0)
        def _():
            out_wait(a - 2, t0, so0)
        transpose(g0, t0, tmp)
        out_store(a, t0, so0)

        gather_wait(idx1, g1, sg1)

        @pl.when(j < _NPAIR - 1)
        def _():
            idx_load(b + 2, idx1, si1)
            idx_wait(a + 2, idx0, si0)
            gather(idx0, g0, sg0)

        @pl.when(j > 0)
        def _():
            out_wait(b - 2, t1, so1)
        transpose(g1, t1, tmp)
        out_store(b, t1, so1)
        return carry

    lax.fori_loop(0, _NPAIR, pair, 0)

    out_wait(_PER_W - 2, t0, so0)
    out_wait(_PER_W - 1, t1, so1)


def kernel(ids, embs, pad):
    del pad
    flat_t = ids.T.reshape(-1).astype(jnp.int32)
    out3 = _gather_kernel(flat_t, embs)
    out5 = out3.reshape(HIST, 4, 128, 8, 128)
    return out5.transpose(2, 4, 0, 1, 3).reshape(BATCH, HIST, EMBED_DIM)
